# baseline (speedup 1.0000x reference)
import functools
import jax
import jax.numpy as jnp
from jax import lax
from jax.experimental import pallas as pl
from jax.experimental.pallas import tpu as pltpu
from jax.experimental.pallas import tpu_sc as plsc

MAXLEN = 200
EMBED = 64
BATCH = 1024
LANES = 16
NC, NS = 2, 16
NW = NC * NS
BB = BATCH // 128
NITEMS = MAXLEN * BB
IPW = NITEMS // NW


@jax.jit
def _run(x_lin, token_table, pos_table):
    mesh = plsc.VectorSubcoreMesh(core_axis_name="c", subcore_axis_name="s")

    scratch = [
        pltpu.VMEM((MAXLEN, EMBED), jnp.float32),
    ]
    scratch += [pltpu.VMEM((128,), jnp.int32) for _ in range(2)]
    scratch += [pltpu.VMEM((128, EMBED), jnp.float32) for _ in range(2)]
    scratch += [pltpu.VMEM((EMBED, 128), jnp.float32) for _ in range(2)]
    scratch += [pltpu.SemaphoreType.DMA for _ in range(6)]

    @functools.partial(
        pl.kernel,
        out_type=jax.ShapeDtypeStruct((MAXLEN * 512, 128), jnp.float32),
        mesh=mesh,
        scratch_types=scratch,
        compiler_params=pltpu.CompilerParams(
            use_tc_tiling_on_sc=False, needs_layout_passes=False
        ),
    )
    def k(xl_hbm, tab_hbm, pos_hbm, out_hbm, pos_v, i0, i1, g0, g1, o0, o1,
          is0, is1, gs0, gs1, os0, os1):
        ibufs, gbufs, obufs = (i0, i1), (g0, g1), (o0, o1)
        isems, gsems, osems = (is0, is1), (gs0, gs1), (os0, os1)

        wid = lax.axis_index("s") * NC + lax.axis_index("c")
        m0 = wid * IPW
        pltpu.sync_copy(pos_hbm, pos_v)

        lanes = lax.iota(jnp.int32, LANES)

        def xrow(m):
            t = m // BB
            j = lax.rem(m, BB)
            return (t // 8) * 64 + j * 8 + lax.rem(t, 8)

        def fire_idx(m, b):
            pltpu.async_copy(xl_hbm.at[xrow(m)], ibufs[b], isems[b])

        def wait_idx(m, b):
            pltpu.make_async_copy(xl_hbm.at[xrow(m)], ibufs[b], isems[b]).wait()

        def fire_gather(b):
            pltpu.async_copy(tab_hbm.at[ibufs[b]], gbufs[b], gsems[b])

        def wait_gather(b):
            pltpu.make_async_copy(tab_hbm.at[ibufs[b]], gbufs[b], gsems[b]).wait()

        def fire_out(m, b):
            t = m // BB
            j = lax.rem(m, BB)
            base = t * 512 + j * 8
            for fo in range(EMBED // 8):
                pltpu.async_copy(
                    obufs[b].at[pl.ds(fo * 8, 8)],
                    out_hbm.at[pl.ds(base + fo * 64, 8)],
                    osems[b],
                )

        def drain_out(b):
            for _ in range(EMBED // 8):
                pltpu.make_async_copy(
                    obufs[b].at[pl.ds(0, 8)],
                    out_hbm.at[pl.ds(0, 8)],
                    osems[b],
                ).wait()

        fire_idx(m0, 0)
        wait_idx(m0, 0)
        fire_gather(0)

        def outer(o, carry):
            for b in range(2):
                l = o * 2 + b
                m = m0 + l

                @pl.when(l + 1 < IPW)
                def _pf_idx():
                    fire_idx(m + 1, 1 - b)

                wait_gather(b)

                @pl.when(l >= 2)
                def _drain():
                    drain_out(b)

                gbuf = gbufs[b]
                obuf = obufs[b]
                t = m // BB

                tcol = lax.broadcast(t, (LANES,))

                @plsc.parallel_loop(0, EMBED, unroll=2)
                def _tr(f):
                    fcol = lax.broadcast(f, (LANES,))
                    s = plsc.load_gather(pos_v, [tcol, fcol])
                    for bb in range(8):
                        rows = lanes + (bb * LANES)
                        v = plsc.load_gather(gbuf, [rows, fcol])
                        obuf[f, pl.ds(bb * LANES, LANES)] = v + s

                fire_out(m, b)

                @pl.when(l + 1 < IPW)
                def _pf_gather():
                    wait_idx(m + 1, 1 - b)
                    fire_gather(1 - b)

            return carry

        lax.fori_loop(0, IPW // 2, outer, 0)
        for b in range(2):
            drain_out(b)

    return k(x_lin, token_table, pos_table)


def kernel(x, token_table, pos_table):
    x_lin = (
        x.T.reshape(25, 8, 8, 128).transpose(0, 2, 1, 3).reshape(1600, 128)
    ).astype(jnp.int32)
    out_lin = _run(x_lin, token_table, pos_table)
    out = (
        out_lin.reshape(MAXLEN, 8, 8, 8, 128)
        .transpose(2, 4, 0, 1, 3)
        .reshape(BATCH, MAXLEN, EMBED)
    )
    return out


# speedup vs baseline: 1.8385x; 1.8385x over previous
import functools
import jax
import jax.numpy as jnp
from jax import lax
from jax.experimental import pallas as pl
from jax.experimental.pallas import tpu as pltpu
from jax.experimental.pallas import tpu_sc as plsc

MAXLEN = 200
EMBED = 64
BATCH = 1024
LANES = 16
NC, NS = 2, 16
NW = NC * NS
BB = BATCH // 128
NITEMS = MAXLEN * BB
IPW = NITEMS // NW


@jax.jit
def _run(x_lin, token_table, pos_table):
    mesh = plsc.VectorSubcoreMesh(core_axis_name="c", subcore_axis_name="s")

    scratch = [
        pltpu.VMEM((MAXLEN, EMBED), jnp.float32),
    ]
    scratch += [pltpu.VMEM((128,), jnp.int32) for _ in range(2)]
    scratch += [pltpu.VMEM((128, EMBED), jnp.float32) for _ in range(2)]
    scratch += [pltpu.VMEM((EMBED, 129), jnp.float32) for _ in range(2)]
    scratch += [pltpu.SemaphoreType.DMA for _ in range(6)]

    @functools.partial(
        pl.kernel,
        out_type=jax.ShapeDtypeStruct((MAXLEN * 512, 128), jnp.float32),
        mesh=mesh,
        scratch_types=scratch,
        compiler_params=pltpu.CompilerParams(
            use_tc_tiling_on_sc=False, needs_layout_passes=False
        ),
    )
    def k(xl_hbm, tab_hbm, pos_hbm, out_hbm, pos_v, i0, i1, g0, g1, o0, o1,
          is0, is1, gs0, gs1, os0, os1):
        ibufs, gbufs, obufs = (i0, i1), (g0, g1), (o0, o1)
        isems, gsems, osems = (is0, is1), (gs0, gs1), (os0, os1)

        wid = lax.axis_index("s") * NC + lax.axis_index("c")
        m0 = wid * IPW
        pltpu.sync_copy(pos_hbm, pos_v)

        lanes = lax.iota(jnp.int32, LANES)

        def xrow(m):
            t = m // BB
            j = lax.rem(m, BB)
            return (t // 8) * 64 + j * 8 + lax.rem(t, 8)

        def fire_idx(m, b):
            pltpu.async_copy(xl_hbm.at[xrow(m)], ibufs[b], isems[b])

        def wait_idx(m, b):
            pltpu.make_async_copy(xl_hbm.at[xrow(m)], ibufs[b], isems[b]).wait()

        def fire_gather(b):
            pltpu.async_copy(tab_hbm.at[ibufs[b]], gbufs[b], gsems[b])

        def wait_gather(b):
            pltpu.make_async_copy(tab_hbm.at[ibufs[b]], gbufs[b], gsems[b]).wait()

        def fire_out(m, b):
            t = m // BB
            j = lax.rem(m, BB)
            base = t * 512 + j * 8
            for fo in range(EMBED // 8):
                pltpu.async_copy(
                    obufs[b].at[pl.ds(fo * 8, 8), pl.ds(0, 128)],
                    out_hbm.at[pl.ds(base + fo * 64, 8)],
                    osems[b],
                )

        def drain_out(b):
            for _ in range(EMBED // 8):
                pltpu.make_async_copy(
                    obufs[b].at[pl.ds(0, 8), pl.ds(0, 128)],
                    out_hbm.at[pl.ds(0, 8)],
                    osems[b],
                ).wait()

        fire_idx(m0, 0)
        wait_idx(m0, 0)
        fire_gather(0)

        def outer(o, carry):
            for b in range(2):
                l = o * 2 + b
                m = m0 + l

                @pl.when(l + 1 < IPW)
                def _pf_idx():
                    fire_idx(m + 1, 1 - b)

                wait_gather(b)

                @pl.when(l >= 2)
                def _drain():
                    drain_out(b)

                gbuf = gbufs[b]
                obuf = obufs[b]
                t = m // BB

                pos4 = [pos_v[t, pl.ds(fg * LANES, LANES)] for fg in range(4)]
                rowidx = [lanes + fg * LANES for fg in range(4)]

                @plsc.parallel_loop(0, 128, unroll=2)
                def _tr(bi):
                    bcol = lax.broadcast(bi, (LANES,))
                    for fg in range(4):
                        v = gbuf[bi, pl.ds(fg * LANES, LANES)]
                        plsc.store_scatter(
                            obuf, [rowidx[fg], bcol], v + pos4[fg]
                        )

                fire_out(m, b)

                @pl.when(l + 1 < IPW)
                def _pf_gather():
                    wait_idx(m + 1, 1 - b)
                    fire_gather(1 - b)

            return carry

        lax.fori_loop(0, IPW // 2, outer, 0)
        for b in range(2):
            drain_out(b)

    return k(x_lin, token_table, pos_table)


def kernel(x, token_table, pos_table):
    x_lin = (
        x.T.reshape(25, 8, 8, 128).transpose(0, 2, 1, 3).reshape(1600, 128)
    ).astype(jnp.int32)
    out_lin = _run(x_lin, token_table, pos_table)
    out = (
        out_lin.reshape(MAXLEN, 8, 8, 8, 128)
        .transpose(2, 4, 0, 1, 3)
        .reshape(BATCH, MAXLEN, EMBED)
    )
    return out


# speedup vs baseline: 2.3606x; 1.2840x over previous
import functools
import jax
import jax.numpy as jnp
from jax import lax
from jax.experimental import pallas as pl
from jax.experimental.pallas import tpu as pltpu
from jax.experimental.pallas import tpu_sc as plsc

MAXLEN = 200
EMBED = 64
BATCH = 1024
LANES = 16
NC, NS = 2, 16
NW = NC * NS
BB = BATCH // 128
NITEMS = MAXLEN * BB
IPW = NITEMS // NW


@jax.jit
def _run(x_lin, token_table, pos_table):
    mesh = plsc.VectorSubcoreMesh(core_axis_name="c", subcore_axis_name="s")

    scratch = [
        pltpu.VMEM((MAXLEN, EMBED), jnp.float32),
        pltpu.VMEM((128, 128), jnp.int32),
    ]
    scratch += [pltpu.VMEM((128, EMBED), jnp.float32) for _ in range(2)]
    scratch += [pltpu.VMEM((EMBED, 129), jnp.float32) for _ in range(2)]
    scratch += [pltpu.SemaphoreType.DMA for _ in range(5)]

    @functools.partial(
        pl.kernel,
        out_type=jax.ShapeDtypeStruct((MAXLEN * 512, 128), jnp.float32),
        mesh=mesh,
        scratch_types=scratch,
        compiler_params=pltpu.CompilerParams(
            use_tc_tiling_on_sc=False, needs_layout_passes=False
        ),
    )
    def k(xl_hbm, tab_hbm, pos_hbm, out_hbm, pos_v, ixs, g0, g1, o0, o1,
          iss, gs0, gs1, os0, os1):
        gbufs, obufs = (g0, g1), (o0, o1)
        gsems, osems = (gs0, gs1), (os0, os1)

        wid = lax.axis_index("s") * NC + lax.axis_index("c")
        m0 = wid * IPW
        kbase = (m0 // 64) * 64
        second = kbase + 128 <= NITEMS
        pltpu.async_copy(
            xl_hbm.at[pl.ds(kbase, 64)], ixs.at[pl.ds(0, 64)], iss
        )

        @pl.when(second)
        def _load2():
            pltpu.async_copy(
                xl_hbm.at[pl.ds(kbase + 64, 64)], ixs.at[pl.ds(64, 64)], iss
            )

        pltpu.sync_copy(pos_hbm, pos_v)
        pltpu.make_async_copy(
            xl_hbm.at[pl.ds(kbase, 64)], ixs.at[pl.ds(0, 64)], iss
        ).wait()

        @pl.when(second)
        def _wait2():
            pltpu.make_async_copy(
                xl_hbm.at[pl.ds(kbase, 64)], ixs.at[pl.ds(0, 64)], iss
            ).wait()

        lanes = lax.iota(jnp.int32, LANES)

        def xrow(m):
            t = m // BB
            j = lax.rem(m, BB)
            return (t // 8) * 64 + j * 8 + lax.rem(t, 8)

        def fire_gather(m, b):
            pltpu.async_copy(
                tab_hbm.at[ixs.at[xrow(m) - kbase]], gbufs[b], gsems[b]
            )

        def wait_gather(m, b):
            pltpu.make_async_copy(
                tab_hbm.at[ixs.at[xrow(m) - kbase]], gbufs[b], gsems[b]
            ).wait()

        def fire_out(m, b):
            t = m // BB
            j = lax.rem(m, BB)
            base = t * 512 + j * 8
            for fo in range(EMBED // 8):
                pltpu.async_copy(
                    obufs[b].at[pl.ds(fo * 8, 8), pl.ds(0, 128)],
                    out_hbm.at[pl.ds(base + fo * 64, 8)],
                    osems[b],
                )

        def drain_out(b):
            for _ in range(EMBED // 8):
                pltpu.make_async_copy(
                    obufs[b].at[pl.ds(0, 8), pl.ds(0, 128)],
                    out_hbm.at[pl.ds(0, 8)],
                    osems[b],
                ).wait()

        fire_gather(m0, 0)
        fire_gather(m0 + 1, 1)

        def outer(o, carry):
            for b in range(2):
                l = o * 2 + b
                m = m0 + l

                wait_gather(m, b)

                @pl.when(l >= 2)
                def _drain():
                    drain_out(b)

                gbuf = gbufs[b]
                obuf = obufs[b]
                t = m // BB

                pos4 = [pos_v[t, pl.ds(fg * LANES, LANES)] for fg in range(4)]
                rowidx = [lanes + fg * LANES for fg in range(4)]

                @plsc.parallel_loop(0, 128, unroll=2)
                def _tr(bi):
                    bcol = lax.broadcast(bi, (LANES,))
                    for fg in range(4):
                        v = gbuf[bi, pl.ds(fg * LANES, LANES)]
                        plsc.store_scatter(
                            obuf, [rowidx[fg], bcol], v + pos4[fg]
                        )

                fire_out(m, b)

                @pl.when(l + 2 < IPW)
                def _pf_gather():
                    fire_gather(m + 2, b)

            return carry

        lax.fori_loop(0, IPW // 2, outer, 0)
        for b in range(2):
            drain_out(b)

    return k(x_lin, token_table, pos_table)


def kernel(x, token_table, pos_table):
    x_lin = (
        x.T.reshape(25, 8, 8, 128).transpose(0, 2, 1, 3).reshape(1600, 128)
    ).astype(jnp.int32)
    out_lin = _run(x_lin, token_table, pos_table)
    out = (
        out_lin.reshape(MAXLEN, 8, 8, 8, 128)
        .transpose(2, 4, 0, 1, 3)
        .reshape(BATCH, MAXLEN, EMBED)
    )
    return out
